# baseline (speedup 1.0000x reference)
import dataclasses
import functools

import jax
import jax.numpy as jnp
from jax import lax
from jax.experimental import pallas as pl
from jax.experimental.pallas import tpu as pltpu
from jax.experimental.pallas import tpu_sc as plsc

NC = 2
NS = 16
NW = NC * NS
K = 128
NWIN = 16
DEG_W = 16


def _sc_mesh():
    return plsc.VectorSubcoreMesh(
        core_axis_name="c", subcore_axis_name="s",
        num_cores=NC, num_subcores=NS)


def _deg_call(dst_r, n_pad, cpt):
    @functools.partial(
        pl.kernel,
        out_type=jax.ShapeDtypeStruct((NW, n_pad), jnp.float32),
        mesh=_sc_mesh(),
        scratch_types=[
            pltpu.VMEM((cpt, K), jnp.int32),
            pltpu.VMEM((n_pad,), jnp.float32),
        ],
        compiler_params=dataclasses.replace(
            pltpu.CompilerParams(), needs_layout_passes=False),
    )
    def k(dst_hbm, out_hbm, idx_v, hist_v):
        core = lax.axis_index("c")
        sub = lax.axis_index("s")
        wid = sub * NC + core

        @pl.loop(0, n_pad // 16)
        def _(i):
            hist_v[pl.ds(i * 16, 16)] = jnp.zeros((16,), jnp.float32)

        pltpu.sync_copy(dst_hbm.at[wid], idx_v)

        ones16 = jnp.ones((16,), jnp.float32)

        @pl.loop(0, cpt)
        def _(c):
            @pl.loop(0, K // 16)
            def _(j):
                v = idx_v[c, pl.ds(j * 16, 16)]
                plsc.addupdate_scatter(hist_v, [v], ones16)

        pltpu.sync_copy(hist_v, out_hbm.at[wid])

    return k(dst_r)


def _agg_call(g_pad, src_r, dst_r, n_pad, cpt, ch, sc_tiling=False):
    stripe = n_pad // NS
    nh = cpt // NWIN

    cp = pltpu.CompilerParams()
    if sc_tiling:
        cp = dataclasses.replace(cp, use_tc_tiling_on_sc=False)

    @functools.partial(
        pl.kernel,
        out_type=jax.ShapeDtypeStruct((NC, n_pad, ch), jnp.float32),
        mesh=_sc_mesh(),
        compiler_params=cp,
        scratch_types=[
            pltpu.VMEM((cpt, K), jnp.int32),
            pltpu.VMEM((2, NWIN, K), jnp.int32),
            pltpu.VMEM((K, ch), jnp.float32),
            pltpu.VMEM((K, ch), jnp.float32),
            pltpu.VMEM_SHARED((n_pad, ch), jnp.float32),
            pltpu.SemaphoreType.DMA,
            pltpu.SemaphoreType.DMA,
            pltpu.SemaphoreType.DMA,
        ],
    )
    def k(g_hbm, src_hbm, dst_hbm, out_hbm, src_v, dst_w, buf_a, buf_b,
          acc, sem_a, sem_b, sem_d):
        core = lax.axis_index("c")
        sub = lax.axis_index("s")
        wid = sub * NC + core
        base = sub * stripe

        @pl.loop(0, K)
        def _(i):
            @pl.loop(0, ch // 16)
            def _(j):
                buf_a[i, pl.ds(j * 16, 16)] = jnp.zeros((16,), jnp.float32)

        for off in range(0, stripe, K):
            rows = min(K, stripe - off)
            pltpu.sync_copy(buf_a.at[pl.ds(0, rows)],
                            acc.at[pl.ds(base + off, rows)])

        pltpu.sync_copy(src_hbm.at[wid], src_v)
        pltpu.async_copy(dst_hbm.at[wid, pl.ds(0, NWIN)], dst_w.at[0], sem_d)
        plsc.subcore_barrier()

        pltpu.async_copy(g_hbm.at[src_v.at[0]], buf_a, sem_a)
        pltpu.async_copy(g_hbm.at[src_v.at[1]], buf_b, sem_b)

        @pl.loop(0, cpt // 2 - 1)
        def _(i):
            c0 = 2 * i
            h = c0 // NWIN
            rel = c0 % NWIN

            @pl.when(rel == 0)
            def _():
                pltpu.make_async_copy(dst_hbm.at[wid, pl.ds(0, NWIN)],
                                      dst_w.at[h % 2], sem_d).wait()

                @pl.when(h + 1 < nh)
                def _():
                    pltpu.async_copy(
                        dst_hbm.at[wid, pl.ds((h + 1) * NWIN, NWIN)],
                        dst_w.at[(h + 1) % 2], sem_d)

            pltpu.make_async_copy(g_hbm.at[src_v.at[c0]], buf_a, sem_a).wait()
            pltpu.sync_copy(buf_a, acc.at[dst_w.at[h % 2, rel]], add=True)
            pltpu.async_copy(g_hbm.at[src_v.at[c0 + 2]], buf_a, sem_a)
            pltpu.make_async_copy(g_hbm.at[src_v.at[c0 + 1]], buf_b,
                                  sem_b).wait()
            pltpu.sync_copy(buf_b, acc.at[dst_w.at[h % 2, rel + 1]], add=True)
            pltpu.async_copy(g_hbm.at[src_v.at[c0 + 3]], buf_b, sem_b)

        pltpu.make_async_copy(g_hbm.at[src_v.at[cpt - 2]], buf_a, sem_a).wait()
        pltpu.sync_copy(buf_a, acc.at[dst_w.at[(nh - 1) % 2, NWIN - 2]],
                        add=True)
        pltpu.make_async_copy(g_hbm.at[src_v.at[cpt - 1]], buf_b, sem_b).wait()
        pltpu.sync_copy(buf_b, acc.at[dst_w.at[(nh - 1) % 2, NWIN - 1]],
                        add=True)

        plsc.subcore_barrier()
        pltpu.sync_copy(acc.at[pl.ds(base, stripe)],
                        out_hbm.at[core, pl.ds(base, stripe)])

    return k(g_pad, src_r, dst_r)


def _prep1_call(degp, x, w, b, n_pad):
    n, cin = x.shape
    ch = w.shape[1]

    def body(d_ref, x_ref, w_ref, b_ref, g_ref, di_ref):
        deg = jnp.sum(d_ref[...], axis=0, keepdims=True) + 1.0
        dinv = lax.rsqrt(deg)
        node = lax.broadcasted_iota(jnp.int32, (1, n_pad), 1)
        dinv = jnp.where(node < n, dinv, 0.0)
        dinv_col = jnp.transpose(dinv)
        h = jnp.dot(x_ref[...].astype(jnp.bfloat16),
                    w_ref[...].astype(jnp.bfloat16),
                    preferred_element_type=jnp.float32) + b_ref[...]
        g = h * dinv_col[:n]
        g_ref[...] = jnp.concatenate(
            [g, jnp.zeros((n_pad - n, ch), jnp.float32)], axis=0)
        di_ref[...] = jnp.broadcast_to(dinv_col, (n_pad, DEG_W))

    return pl.pallas_call(
        body,
        out_shape=[
            jax.ShapeDtypeStruct((n_pad, ch), jnp.float32),
            jax.ShapeDtypeStruct((n_pad, DEG_W), jnp.float32),
        ],
    )(degp, x, w, b.reshape(1, ch))


def _g2_call(p1, g1, dinv, w2, b2, stripe):
    n_pad, ch = g1.shape
    co = w2.shape[1]

    def body(p_ref, g_ref, di_ref, w_ref, b_ref, x1_ref, g2_ref):
        p = p_ref[...]
        dinv = di_ref[...][:, 0:1]
        x1 = jnp.maximum((p[0] + p[1] + g_ref[...]) * dinv, 0.0)
        x1_ref[...] = x1
        h2 = jnp.dot(x1.astype(jnp.bfloat16),
                     w_ref[...].astype(jnp.bfloat16),
                     preferred_element_type=jnp.float32) + b_ref[...]
        g2_ref[...] = h2 * dinv

    return pl.pallas_call(
        body,
        grid=(n_pad // stripe,),
        in_specs=[
            pl.BlockSpec((2, stripe, ch), lambda i: (0, i, 0)),
            pl.BlockSpec((stripe, ch), lambda i: (i, 0)),
            pl.BlockSpec((stripe, DEG_W), lambda i: (i, 0)),
            pl.BlockSpec((ch, co), lambda i: (0, 0)),
            pl.BlockSpec((1, co), lambda i: (0, 0)),
        ],
        out_specs=[
            pl.BlockSpec((stripe, ch), lambda i: (i, 0)),
            pl.BlockSpec((stripe, co), lambda i: (i, 0)),
        ],
        out_shape=[
            jax.ShapeDtypeStruct((n_pad, ch), jnp.float32),
            jax.ShapeDtypeStruct((n_pad, co), jnp.float32),
        ],
    )(p1, g1, dinv, w2, b2.reshape(1, co))


def _comb2_call(p2, g2, dinv, stripe):
    n_pad, co = g2.shape

    def body(p_ref, g_ref, di_ref, x2_ref):
        p = p_ref[...]
        dinv = di_ref[...][:, 0:1]
        x2_ref[...] = (p[0] + p[1] + g_ref[...]) * dinv

    return pl.pallas_call(
        body,
        grid=(n_pad // stripe,),
        in_specs=[
            pl.BlockSpec((2, stripe, co), lambda i: (0, i, 0)),
            pl.BlockSpec((stripe, co), lambda i: (i, 0)),
            pl.BlockSpec((stripe, DEG_W), lambda i: (i, 0)),
        ],
        out_specs=pl.BlockSpec((stripe, co), lambda i: (i, 0)),
        out_shape=jax.ShapeDtypeStruct((n_pad, co), jnp.float32),
    )(p2, g2, dinv)


def kernel(X, edge_index, W1, b1, W2, b2):
    n, _ = X.shape
    e = edge_index.shape[1]

    stripe = -(-(n + 1) // NS)
    stripe = -(-stripe // 8) * 8
    n_pad = NS * stripe

    cpt = -(-e // (NW * K))
    cpt = -(-cpt // NWIN) * NWIN
    ep = cpt * NW * K

    src = edge_index[0].astype(jnp.int32)
    dst = edge_index[1].astype(jnp.int32)
    pad = ep - e
    pad_rows = n_pad - n
    pad_idx = jnp.arange(pad, dtype=jnp.int32)
    src_p = jnp.concatenate([src, n + pad_idx % pad_rows])
    dst_p = jnp.concatenate([dst, n + (pad_idx * 7 + 3) % pad_rows])
    src_r = src_p.reshape(NW, cpt, K)
    dst_r = dst_p.reshape(NW, cpt, K)

    degp = _deg_call(dst_r, n_pad, cpt)
    g1, dinv = _prep1_call(degp, X, W1, b1, n_pad)
    p1 = _agg_call(g1, src_r, dst_r, n_pad, cpt, W1.shape[1])
    x1, g2 = _g2_call(p1, g1, dinv, W2, b2, stripe)
    p2 = _agg_call(g2, src_r, dst_r, n_pad, cpt, W2.shape[1],
                   sc_tiling=True)
    x2 = _comb2_call(p2, g2, dinv, stripe)

    return (x1[:n], x2[:n])


# speedup vs baseline: 1.0061x; 1.0061x over previous
import dataclasses
import functools

import jax
import jax.numpy as jnp
from jax import lax
from jax.experimental import pallas as pl
from jax.experimental.pallas import tpu as pltpu
from jax.experimental.pallas import tpu_sc as plsc

NC = 2
NS = 16
NW = NC * NS
K = 128
NWIN = 16
DEG_W = 16


def _sc_mesh():
    return plsc.VectorSubcoreMesh(
        core_axis_name="c", subcore_axis_name="s",
        num_cores=NC, num_subcores=NS)


def _deg_call(dst_r, n_pad, cpt):
    @functools.partial(
        pl.kernel,
        out_type=jax.ShapeDtypeStruct((NW, n_pad), jnp.float32),
        mesh=_sc_mesh(),
        scratch_types=[
            pltpu.VMEM((cpt, K), jnp.int32),
            pltpu.VMEM((n_pad,), jnp.float32),
        ],
        compiler_params=dataclasses.replace(
            pltpu.CompilerParams(), needs_layout_passes=False),
    )
    def k(dst_hbm, out_hbm, idx_v, hist_v):
        core = lax.axis_index("c")
        sub = lax.axis_index("s")
        wid = sub * NC + core

        @pl.loop(0, n_pad // 16)
        def _(i):
            hist_v[pl.ds(i * 16, 16)] = jnp.zeros((16,), jnp.float32)

        pltpu.sync_copy(dst_hbm.at[wid], idx_v)

        ones16 = jnp.ones((16,), jnp.float32)

        @pl.loop(0, cpt)
        def _(c):
            @pl.loop(0, K // 16)
            def _(j):
                v = idx_v[c, pl.ds(j * 16, 16)]
                plsc.addupdate_scatter(hist_v, [v], ones16)

        pltpu.sync_copy(hist_v, out_hbm.at[wid])

    return k(dst_r)


def _agg_call(g_pad, src_r, dst_r, n_pad, cpt, ch, sc_tiling=False):
    stripe = n_pad // NS
    nh = cpt // NWIN

    cp = pltpu.CompilerParams()
    if sc_tiling:
        cp = dataclasses.replace(cp, use_tc_tiling_on_sc=False)

    @functools.partial(
        pl.kernel,
        out_type=jax.ShapeDtypeStruct((NC, n_pad, ch), jnp.float32),
        mesh=_sc_mesh(),
        compiler_params=cp,
        scratch_types=[
            pltpu.VMEM((cpt, K), jnp.int32),
            pltpu.VMEM((2, NWIN, K), jnp.int32),
            pltpu.VMEM((K, ch), jnp.float32),
            pltpu.VMEM((K, ch), jnp.float32),
            pltpu.VMEM_SHARED((n_pad, ch), jnp.float32),
            pltpu.SemaphoreType.DMA,
            pltpu.SemaphoreType.DMA,
            pltpu.SemaphoreType.DMA,
        ],
    )
    def k(g_hbm, src_hbm, dst_hbm, out_hbm, src_v, dst_w, buf_a, buf_b,
          acc, sem_a, sem_b, sem_d):
        core = lax.axis_index("c")
        sub = lax.axis_index("s")
        wid = sub * NC + core
        base = sub * stripe

        pltpu.async_copy(src_hbm.at[wid], src_v, sem_a)
        pltpu.async_copy(dst_hbm.at[wid, pl.ds(0, NWIN)], dst_w.at[0], sem_d)

        @pl.loop(0, K)
        def _(i):
            @pl.loop(0, ch // 16)
            def _(j):
                buf_a[i, pl.ds(j * 16, 16)] = jnp.zeros((16,), jnp.float32)

        for off in range(0, stripe, K):
            rows = min(K, stripe - off)
            pltpu.sync_copy(buf_a.at[pl.ds(0, rows)],
                            acc.at[pl.ds(base + off, rows)])

        pltpu.make_async_copy(src_hbm.at[wid], src_v, sem_a).wait()
        plsc.subcore_barrier()

        pltpu.async_copy(g_hbm.at[src_v.at[0]], buf_a, sem_a)
        pltpu.async_copy(g_hbm.at[src_v.at[1]], buf_b, sem_b)

        @pl.loop(0, cpt // 2 - 1)
        def _(i):
            c0 = 2 * i
            h = c0 // NWIN
            rel = c0 % NWIN

            @pl.when(rel == 0)
            def _():
                pltpu.make_async_copy(dst_hbm.at[wid, pl.ds(0, NWIN)],
                                      dst_w.at[h % 2], sem_d).wait()

                @pl.when(h + 1 < nh)
                def _():
                    pltpu.async_copy(
                        dst_hbm.at[wid, pl.ds((h + 1) * NWIN, NWIN)],
                        dst_w.at[(h + 1) % 2], sem_d)

            pltpu.make_async_copy(g_hbm.at[src_v.at[c0]], buf_a, sem_a).wait()
            pltpu.sync_copy(buf_a, acc.at[dst_w.at[h % 2, rel]], add=True)
            pltpu.async_copy(g_hbm.at[src_v.at[c0 + 2]], buf_a, sem_a)
            pltpu.make_async_copy(g_hbm.at[src_v.at[c0 + 1]], buf_b,
                                  sem_b).wait()
            pltpu.sync_copy(buf_b, acc.at[dst_w.at[h % 2, rel + 1]], add=True)
            pltpu.async_copy(g_hbm.at[src_v.at[c0 + 3]], buf_b, sem_b)

        pltpu.make_async_copy(g_hbm.at[src_v.at[cpt - 2]], buf_a, sem_a).wait()
        pltpu.sync_copy(buf_a, acc.at[dst_w.at[(nh - 1) % 2, NWIN - 2]],
                        add=True)
        pltpu.make_async_copy(g_hbm.at[src_v.at[cpt - 1]], buf_b, sem_b).wait()
        pltpu.sync_copy(buf_b, acc.at[dst_w.at[(nh - 1) % 2, NWIN - 1]],
                        add=True)

        plsc.subcore_barrier()
        pltpu.sync_copy(acc.at[pl.ds(base, stripe)],
                        out_hbm.at[core, pl.ds(base, stripe)])

    return k(g_pad, src_r, dst_r)


def _prep1_call(degp, x, w, b, n_pad):
    n, cin = x.shape
    ch = w.shape[1]

    def body(d_ref, x_ref, w_ref, b_ref, g_ref, di_ref):
        deg = jnp.sum(d_ref[...], axis=0, keepdims=True) + 1.0
        dinv = lax.rsqrt(deg)
        node = lax.broadcasted_iota(jnp.int32, (1, n_pad), 1)
        dinv = jnp.where(node < n, dinv, 0.0)
        dinv_col = jnp.transpose(dinv)
        h = jnp.dot(x_ref[...], w_ref[...],
                    preferred_element_type=jnp.float32) + b_ref[...]
        g = h * dinv_col[:n]
        g_ref[...] = jnp.concatenate(
            [g, jnp.zeros((n_pad - n, ch), jnp.float32)], axis=0)
        di_ref[...] = jnp.broadcast_to(dinv_col, (n_pad, DEG_W))

    return pl.pallas_call(
        body,
        out_shape=[
            jax.ShapeDtypeStruct((n_pad, ch), jnp.float32),
            jax.ShapeDtypeStruct((n_pad, DEG_W), jnp.float32),
        ],
    )(degp, x, w, b.reshape(1, ch))


def _g2_call(p1, g1, dinv, w2, b2, stripe):
    n_pad, ch = g1.shape
    co = w2.shape[1]

    def body(p_ref, g_ref, di_ref, w_ref, b_ref, x1_ref, g2_ref):
        p = p_ref[...]
        dinv = di_ref[...][:, 0:1]
        x1 = jnp.maximum((p[0] + p[1] + g_ref[...]) * dinv, 0.0)
        x1_ref[...] = x1
        h2 = jnp.dot(x1, w_ref[...],
                     preferred_element_type=jnp.float32) + b_ref[...]
        g2_ref[...] = h2 * dinv

    return pl.pallas_call(
        body,
        grid=(n_pad // stripe,),
        in_specs=[
            pl.BlockSpec((2, stripe, ch), lambda i: (0, i, 0)),
            pl.BlockSpec((stripe, ch), lambda i: (i, 0)),
            pl.BlockSpec((stripe, DEG_W), lambda i: (i, 0)),
            pl.BlockSpec((ch, co), lambda i: (0, 0)),
            pl.BlockSpec((1, co), lambda i: (0, 0)),
        ],
        out_specs=[
            pl.BlockSpec((stripe, ch), lambda i: (i, 0)),
            pl.BlockSpec((stripe, co), lambda i: (i, 0)),
        ],
        out_shape=[
            jax.ShapeDtypeStruct((n_pad, ch), jnp.float32),
            jax.ShapeDtypeStruct((n_pad, co), jnp.float32),
        ],
    )(p1, g1, dinv, w2, b2.reshape(1, co))


def _comb2_call(p2, g2, dinv, stripe):
    n_pad, co = g2.shape

    def body(p_ref, g_ref, di_ref, x2_ref):
        p = p_ref[...]
        dinv = di_ref[...][:, 0:1]
        x2_ref[...] = (p[0] + p[1] + g_ref[...]) * dinv

    return pl.pallas_call(
        body,
        grid=(n_pad // stripe,),
        in_specs=[
            pl.BlockSpec((2, stripe, co), lambda i: (0, i, 0)),
            pl.BlockSpec((stripe, co), lambda i: (i, 0)),
            pl.BlockSpec((stripe, DEG_W), lambda i: (i, 0)),
        ],
        out_specs=pl.BlockSpec((stripe, co), lambda i: (i, 0)),
        out_shape=jax.ShapeDtypeStruct((n_pad, co), jnp.float32),
    )(p2, g2, dinv)


def kernel(X, edge_index, W1, b1, W2, b2):
    n, _ = X.shape
    e = edge_index.shape[1]

    stripe = -(-(n + 1) // NS)
    stripe = -(-stripe // 8) * 8
    n_pad = NS * stripe

    cpt = -(-e // (NW * K))
    cpt = -(-cpt // NWIN) * NWIN
    ep = cpt * NW * K

    src = edge_index[0].astype(jnp.int32)
    dst = edge_index[1].astype(jnp.int32)
    pad = ep - e
    pad_rows = n_pad - n
    pad_idx = jnp.arange(pad, dtype=jnp.int32)
    src_p = jnp.concatenate([src, n + pad_idx % pad_rows])
    dst_p = jnp.concatenate([dst, n + (pad_idx * 7 + 3) % pad_rows])
    src_r = src_p.reshape(NW, cpt, K)
    dst_r = dst_p.reshape(NW, cpt, K)

    degp = _deg_call(dst_r, n_pad, cpt)
    g1, dinv = _prep1_call(degp, X, W1, b1, n_pad)
    p1 = _agg_call(g1, src_r, dst_r, n_pad, cpt, W1.shape[1])
    x1, g2 = _g2_call(p1, g1, dinv, W2, b2, stripe)
    p2 = _agg_call(g2, src_r, dst_r, n_pad, cpt, W2.shape[1],
                   sc_tiling=True)
    x2 = _comb2_call(p2, g2, dinv, stripe)

    return (x1[:n], x2[:n])


# speedup vs baseline: 1.0064x; 1.0002x over previous
import dataclasses
import functools

import jax
import jax.numpy as jnp
from jax import lax
from jax.experimental import pallas as pl
from jax.experimental.pallas import tpu as pltpu
from jax.experimental.pallas import tpu_sc as plsc

NC = 2
NS = 16
NW = NC * NS
K = 128
NWIN = 16
DEG_W = 16

def _tc_params():
    return pltpu.CompilerParams(skip_device_barrier=True)


def _sc_mesh():
    return plsc.VectorSubcoreMesh(
        core_axis_name="c", subcore_axis_name="s",
        num_cores=NC, num_subcores=NS)


def _deg_call(dst_r, n_pad, cpt):
    @functools.partial(
        pl.kernel,
        out_type=jax.ShapeDtypeStruct((NW, n_pad), jnp.float32),
        mesh=_sc_mesh(),
        scratch_types=[
            pltpu.VMEM((cpt, K), jnp.int32),
            pltpu.VMEM((n_pad,), jnp.float32),
        ],
        compiler_params=dataclasses.replace(
            pltpu.CompilerParams(skip_device_barrier=True),
            needs_layout_passes=False),
    )
    def k(dst_hbm, out_hbm, idx_v, hist_v):
        core = lax.axis_index("c")
        sub = lax.axis_index("s")
        wid = sub * NC + core

        @pl.loop(0, n_pad // 16)
        def _(i):
            hist_v[pl.ds(i * 16, 16)] = jnp.zeros((16,), jnp.float32)

        pltpu.sync_copy(dst_hbm.at[wid], idx_v)

        ones16 = jnp.ones((16,), jnp.float32)

        @pl.loop(0, cpt)
        def _(c):
            @pl.loop(0, K // 16)
            def _(j):
                v = idx_v[c, pl.ds(j * 16, 16)]
                plsc.addupdate_scatter(hist_v, [v], ones16)

        pltpu.sync_copy(hist_v, out_hbm.at[wid])

    return k(dst_r)


def _agg_call(g_pad, src_r, dst_r, n_pad, cpt, ch, sc_tiling=False):
    stripe = n_pad // NS
    nh = cpt // NWIN

    cp = pltpu.CompilerParams(skip_device_barrier=True)
    if sc_tiling:
        cp = dataclasses.replace(cp, use_tc_tiling_on_sc=False)

    @functools.partial(
        pl.kernel,
        out_type=jax.ShapeDtypeStruct((NC, n_pad, ch), jnp.float32),
        mesh=_sc_mesh(),
        compiler_params=cp,
        scratch_types=[
            pltpu.VMEM((cpt, K), jnp.int32),
            pltpu.VMEM((2, NWIN, K), jnp.int32),
            pltpu.VMEM((K, ch), jnp.float32),
            pltpu.VMEM((K, ch), jnp.float32),
            pltpu.VMEM_SHARED((n_pad, ch), jnp.float32),
            pltpu.SemaphoreType.DMA,
            pltpu.SemaphoreType.DMA,
            pltpu.SemaphoreType.DMA,
        ],
    )
    def k(g_hbm, src_hbm, dst_hbm, out_hbm, src_v, dst_w, buf_a, buf_b,
          acc, sem_a, sem_b, sem_d):
        core = lax.axis_index("c")
        sub = lax.axis_index("s")
        wid = sub * NC + core
        base = sub * stripe

        pltpu.async_copy(src_hbm.at[wid], src_v, sem_a)
        pltpu.async_copy(dst_hbm.at[wid, pl.ds(0, NWIN)], dst_w.at[0], sem_d)

        @pl.loop(0, K)
        def _(i):
            @pl.loop(0, ch // 16)
            def _(j):
                buf_a[i, pl.ds(j * 16, 16)] = jnp.zeros((16,), jnp.float32)

        for off in range(0, stripe, K):
            rows = min(K, stripe - off)
            pltpu.sync_copy(buf_a.at[pl.ds(0, rows)],
                            acc.at[pl.ds(base + off, rows)])

        pltpu.make_async_copy(src_hbm.at[wid], src_v, sem_a).wait()
        plsc.subcore_barrier()

        pltpu.async_copy(g_hbm.at[src_v.at[0]], buf_a, sem_a)
        pltpu.async_copy(g_hbm.at[src_v.at[1]], buf_b, sem_b)

        @pl.loop(0, cpt // 2 - 1)
        def _(i):
            c0 = 2 * i
            h = c0 // NWIN
            rel = c0 % NWIN

            @pl.when(rel == 0)
            def _():
                pltpu.make_async_copy(dst_hbm.at[wid, pl.ds(0, NWIN)],
                                      dst_w.at[h % 2], sem_d).wait()

                @pl.when(h + 1 < nh)
                def _():
                    pltpu.async_copy(
                        dst_hbm.at[wid, pl.ds((h + 1) * NWIN, NWIN)],
                        dst_w.at[(h + 1) % 2], sem_d)

            pltpu.make_async_copy(g_hbm.at[src_v.at[c0]], buf_a, sem_a).wait()
            pltpu.sync_copy(buf_a, acc.at[dst_w.at[h % 2, rel]], add=True)
            pltpu.async_copy(g_hbm.at[src_v.at[c0 + 2]], buf_a, sem_a)
            pltpu.make_async_copy(g_hbm.at[src_v.at[c0 + 1]], buf_b,
                                  sem_b).wait()
            pltpu.sync_copy(buf_b, acc.at[dst_w.at[h % 2, rel + 1]], add=True)
            pltpu.async_copy(g_hbm.at[src_v.at[c0 + 3]], buf_b, sem_b)

        pltpu.make_async_copy(g_hbm.at[src_v.at[cpt - 2]], buf_a, sem_a).wait()
        pltpu.sync_copy(buf_a, acc.at[dst_w.at[(nh - 1) % 2, NWIN - 2]],
                        add=True)
        pltpu.make_async_copy(g_hbm.at[src_v.at[cpt - 1]], buf_b, sem_b).wait()
        pltpu.sync_copy(buf_b, acc.at[dst_w.at[(nh - 1) % 2, NWIN - 1]],
                        add=True)

        plsc.subcore_barrier()
        pltpu.sync_copy(acc.at[pl.ds(base, stripe)],
                        out_hbm.at[core, pl.ds(base, stripe)])

    return k(g_pad, src_r, dst_r)


def _prep1_call(degp, x, w, b, n_pad):
    n, cin = x.shape
    ch = w.shape[1]

    def body(d_ref, x_ref, w_ref, b_ref, g_ref, di_ref):
        deg = jnp.sum(d_ref[...], axis=0, keepdims=True) + 1.0
        dinv = lax.rsqrt(deg)
        node = lax.broadcasted_iota(jnp.int32, (1, n_pad), 1)
        dinv = jnp.where(node < n, dinv, 0.0)
        dinv_col = jnp.transpose(dinv)
        h = jnp.dot(x_ref[...], w_ref[...],
                    preferred_element_type=jnp.float32) + b_ref[...]
        g = h * dinv_col[:n]
        g_ref[...] = jnp.concatenate(
            [g, jnp.zeros((n_pad - n, ch), jnp.float32)], axis=0)
        di_ref[...] = jnp.broadcast_to(dinv_col, (n_pad, DEG_W))

    return pl.pallas_call(
        body,
        compiler_params=_tc_params(),
        out_shape=[
            jax.ShapeDtypeStruct((n_pad, ch), jnp.float32),
            jax.ShapeDtypeStruct((n_pad, DEG_W), jnp.float32),
        ],
    )(degp, x, w, b.reshape(1, ch))


def _g2_call(p1, g1, dinv, w2, b2, stripe):
    n_pad, ch = g1.shape
    co = w2.shape[1]

    def body(p_ref, g_ref, di_ref, w_ref, b_ref, x1_ref, g2_ref):
        p = p_ref[...]
        dinv = di_ref[...][:, 0:1]
        x1 = jnp.maximum((p[0] + p[1] + g_ref[...]) * dinv, 0.0)
        x1_ref[...] = x1
        h2 = jnp.dot(x1, w_ref[...],
                     preferred_element_type=jnp.float32) + b_ref[...]
        g2_ref[...] = h2 * dinv

    return pl.pallas_call(
        body,
        grid=(n_pad // stripe,),
        compiler_params=_tc_params(),
        in_specs=[
            pl.BlockSpec((2, stripe, ch), lambda i: (0, i, 0)),
            pl.BlockSpec((stripe, ch), lambda i: (i, 0)),
            pl.BlockSpec((stripe, DEG_W), lambda i: (i, 0)),
            pl.BlockSpec((ch, co), lambda i: (0, 0)),
            pl.BlockSpec((1, co), lambda i: (0, 0)),
        ],
        out_specs=[
            pl.BlockSpec((stripe, ch), lambda i: (i, 0)),
            pl.BlockSpec((stripe, co), lambda i: (i, 0)),
        ],
        out_shape=[
            jax.ShapeDtypeStruct((n_pad, ch), jnp.float32),
            jax.ShapeDtypeStruct((n_pad, co), jnp.float32),
        ],
    )(p1, g1, dinv, w2, b2.reshape(1, co))


def _comb2_call(p2, g2, dinv, stripe):
    n_pad, co = g2.shape

    def body(p_ref, g_ref, di_ref, x2_ref):
        p = p_ref[...]
        dinv = di_ref[...][:, 0:1]
        x2_ref[...] = (p[0] + p[1] + g_ref[...]) * dinv

    return pl.pallas_call(
        body,
        grid=(n_pad // stripe,),
        compiler_params=_tc_params(),
        in_specs=[
            pl.BlockSpec((2, stripe, co), lambda i: (0, i, 0)),
            pl.BlockSpec((stripe, co), lambda i: (i, 0)),
            pl.BlockSpec((stripe, DEG_W), lambda i: (i, 0)),
        ],
        out_specs=pl.BlockSpec((stripe, co), lambda i: (i, 0)),
        out_shape=jax.ShapeDtypeStruct((n_pad, co), jnp.float32),
    )(p2, g2, dinv)


def kernel(X, edge_index, W1, b1, W2, b2):
    n, _ = X.shape
    e = edge_index.shape[1]

    stripe = -(-(n + 1) // NS)
    stripe = -(-stripe // 8) * 8
    n_pad = NS * stripe

    cpt = -(-e // (NW * K))
    cpt = -(-cpt // NWIN) * NWIN
    ep = cpt * NW * K

    src = edge_index[0].astype(jnp.int32)
    dst = edge_index[1].astype(jnp.int32)
    pad = ep - e
    pad_rows = n_pad - n
    pad_idx = jnp.arange(pad, dtype=jnp.int32)
    src_p = jnp.concatenate([src, n + pad_idx % pad_rows])
    dst_p = jnp.concatenate([dst, n + (pad_idx * 7 + 3) % pad_rows])
    src_r = src_p.reshape(NW, cpt, K)
    dst_r = dst_p.reshape(NW, cpt, K)

    degp = _deg_call(dst_r, n_pad, cpt)
    g1, dinv = _prep1_call(degp, X, W1, b1, n_pad)
    p1 = _agg_call(g1, src_r, dst_r, n_pad, cpt, W1.shape[1])
    x1, g2 = _g2_call(p1, g1, dinv, W2, b2, stripe)
    p2 = _agg_call(g2, src_r, dst_r, n_pad, cpt, W2.shape[1],
                   sc_tiling=True)
    x2 = _comb2_call(p2, g2, dinv, stripe)

    return (x1[:n], x2[:n])
